# Initial kernel scaffold; baseline (speedup 1.0000x reference)
import jax
import jax.numpy as jnp
from jax.experimental import pallas as pl


def kernel(x, edge_index):
    raise NotImplementedError("write your pallas kernel here")


# speedup vs baseline: 24.2100x; 24.2100x over previous
import functools

import jax
import jax.numpy as jnp
from jax import lax
from jax.experimental import pallas as pl
from jax.experimental.pallas import tpu as pltpu
from jax.experimental.pallas import tpu_sc as plsc


def _round_up(a, b):
    return (a + b - 1) // b * b


_CP = pltpu.CompilerParams(needs_layout_passes=False, use_tc_tiling_on_sc=False)


def _wid(nc):
    return lax.axis_index("s") * nc + lax.axis_index("c")


def _make_count_kernels(E, N, NW, NC, mesh):
    CH = _round_up(-(-E // NW), 128)
    EP = CH * NW
    NH = _round_up(N + 16, NW * 16)
    COLS = NH // NW
    IT = CH // 16

    def body_a(idxj_hbm, local_hbm, hist_hbm, idx_v, counts_v, local_v):
        w = _wid(NC)
        pltpu.sync_copy(idxj_hbm.at[pl.ds(w * CH, CH)], idx_v)

        def zero(j, c):
            counts_v[pl.ds(j * 16, 16)] = jnp.zeros((16,), jnp.int32)
            return c
        lax.fori_loop(jnp.int32(0), jnp.int32(NH // 16), zero, 0)

        def step(i, c):
            v = idx_v[pl.ds(i * 16, 16)]
            cnt, last = plsc.scan_count(v)
            base = plsc.load_gather(counts_v, [v])
            local_v[pl.ds(i * 16, 16)] = base + cnt - 1
            plsc.store_scatter(counts_v, [v], base + cnt, mask=last)
            return c
        lax.fori_loop(jnp.int32(0), jnp.int32(IT), step, 0)

        pltpu.sync_copy(local_v, local_hbm.at[pl.ds(w * CH, CH)])
        pltpu.sync_copy(counts_v, hist_hbm.at[w])

    kernel_a = pl.kernel(
        body_a,
        out_type=(jax.ShapeDtypeStruct((EP,), jnp.int32),
                  jax.ShapeDtypeStruct((NW, NH), jnp.int32)),
        mesh=mesh,
        compiler_params=_CP,
        scratch_types=[pltpu.VMEM((CH,), jnp.int32),
                       pltpu.VMEM((NH,), jnp.int32),
                       pltpu.VMEM((CH,), jnp.int32)],
    )

    def body_b(hist_hbm, offs_hbm, blk_v):
        w = _wid(NC)
        pltpu.sync_copy(hist_hbm.at[:, pl.ds(w * COLS, COLS)], blk_v)
        for j in range(COLS // 16):
            def step(t, acc):
                new = blk_v[t, pl.ds(j * 16, 16)]
                blk_v[t, pl.ds(j * 16, 16)] = acc
                return acc + new
            lax.fori_loop(jnp.int32(0), jnp.int32(NW), step, jnp.zeros((16,), jnp.int32))
        pltpu.sync_copy(blk_v, offs_hbm.at[:, pl.ds(w * COLS, COLS)])

    kernel_b = pl.kernel(
        body_b,
        out_type=jax.ShapeDtypeStruct((NW, NH), jnp.int32),
        mesh=mesh,
        compiler_params=_CP,
        scratch_types=[pltpu.VMEM((NW, COLS), jnp.int32)],
    )

    def body_c(idxj_hbm, local_hbm, offs_hbm, out_hbm,
               idx_v, local_v, offrow_v, out_v):
        w = _wid(NC)
        pltpu.sync_copy(offs_hbm.at[w], offrow_v)
        pltpu.sync_copy(idxj_hbm.at[pl.ds(w * CH, CH)], idx_v)
        pltpu.sync_copy(local_hbm.at[pl.ds(w * CH, CH)], local_v)

        def step(i, c):
            v = idx_v[pl.ds(i * 16, 16)]
            out_v[pl.ds(i * 16, 16)] = (
                local_v[pl.ds(i * 16, 16)] + plsc.load_gather(offrow_v, [v]))
            return c
        lax.fori_loop(jnp.int32(0), jnp.int32(IT), step, 0)
        pltpu.sync_copy(out_v, out_hbm.at[pl.ds(w * CH, CH)])

    kernel_c = pl.kernel(
        body_c,
        out_type=jax.ShapeDtypeStruct((EP,), jnp.int32),
        mesh=mesh,
        compiler_params=_CP,
        scratch_types=[pltpu.VMEM((CH,), jnp.int32),
                       pltpu.VMEM((CH,), jnp.int32),
                       pltpu.VMEM((NH,), jnp.int32),
                       pltpu.VMEM((CH,), jnp.int32)],
    )
    return kernel_a, kernel_b, kernel_c, CH, EP, NH


def _make_gather_kernel(E, N, D, NW, NC, mesh):
    CH = _round_up(-(-E // NW), 128)
    EP = CH * NW
    G = 128
    NG = CH // G

    def body(x_hbm, idxi_hbm, data_hbm, idx_v, rows_v, sem):
        w = _wid(NC)
        pltpu.sync_copy(idxi_hbm.at[pl.ds(w * CH, CH)], idx_v)

        def step(g, c):
            pltpu.async_copy(
                x_hbm.at[idx_v.at[pl.ds(g * G, G)]], rows_v, sem).wait()
            pltpu.sync_copy(rows_v, data_hbm.at[pl.ds(w * CH + g * G, G)])
            return c
        lax.fori_loop(jnp.int32(0), jnp.int32(NG), step, 0)

    return pl.kernel(
        body,
        out_type=jax.ShapeDtypeStruct((EP, D), jnp.float32),
        mesh=mesh,
        compiler_params=_CP,
        scratch_types=[pltpu.VMEM((CH,), jnp.int32),
                       pltpu.VMEM((G, D), jnp.float32),
                       pltpu.SemaphoreType.DMA],
    )


def kernel(x, edge_index):
    N, D = x.shape
    E = edge_index.shape[0]
    info = plsc.get_sparse_core_info()
    NC, NS = info.num_cores, info.num_subcores
    NW = NC * NS
    mesh = plsc.VectorSubcoreMesh(core_axis_name="c", subcore_axis_name="s")

    kernel_a, kernel_b, kernel_c, CH, EP, NH = _make_count_kernels(
        E, N, NW, NC, mesh)
    kernel_d = _make_gather_kernel(E, N, D, NW, NC, mesh)

    idxi = edge_index[:, 0].astype(jnp.int32)
    idxj = edge_index[:, 1].astype(jnp.int32)
    pad = EP - E
    idxi_p = jnp.concatenate([idxi, jnp.zeros((pad,), jnp.int32)])
    idxj_p = jnp.concatenate([idxj, jnp.full((pad,), N, jnp.int32)])

    local, hist = kernel_a(idxj_p)
    offs = kernel_b(hist)
    idx_el = kernel_c(idxj_p, local, offs)[:E]
    data = kernel_d(x, idxi_p)[:E]

    idxs = jnp.concatenate(
        [edge_index[:, 1][None, :], idx_el.astype(edge_index.dtype)[None, :]],
        axis=0)
    return data, idxs


# speedup vs baseline: 26.2106x; 1.0826x over previous
import jax
import jax.numpy as jnp
from jax import lax
from jax.experimental import pallas as pl
from jax.experimental.pallas import tpu as pltpu
from jax.experimental.pallas import tpu_sc as plsc


def _round_up(a, b):
    return (a + b - 1) // b * b


_CP = pltpu.CompilerParams(needs_layout_passes=False, use_tc_tiling_on_sc=False)

_G = 128


def _wid(nc):
    return lax.axis_index("s") * nc + lax.axis_index("c")


def _gather_ring(x_hbm, data_hbm, idxi_v, rows, semg, semw, base, g_lo, g_hi):
    def fire_gather(g, b):
        pltpu.async_copy(
            x_hbm.at[idxi_v.at[pl.ds(g * _G, _G)]], rows[b], semg[b])

    def wait_gather(b):
        pltpu.make_async_copy(
            x_hbm.at[idxi_v.at[pl.ds(0, _G)]], rows[b], semg[b]).wait()

    def wb_slice(g):
        return data_hbm.at[pl.ds(base + g * _G, _G)]

    def prologue():
        fire_gather(jnp.int32(g_lo), 0)
        if g_lo + 1 < g_hi:
            fire_gather(jnp.int32(g_lo + 1), 1)

    def ring():
        def step(g2, c):
            for b in range(2):
                g = g2 * 2 + b
                @pl.when(g < g_hi)
                def _():
                    wait_gather(b)
                    pltpu.async_copy(rows[b], wb_slice(g), semw[b])
                    pltpu.make_async_copy(rows[b], wb_slice(g), semw[b]).wait()
                    @pl.when(g + 2 < g_hi)
                    def _():
                        fire_gather(g + 2, b)
            return c
        lax.fori_loop(jnp.int32(g_lo // 2), jnp.int32((g_hi + 1) // 2), step, 0)

    return prologue, ring


def _make_kernels(E, N, D, NW, NC, mesh):
    CH = _round_up(-(-E // NW), 2 * _G)
    EP = CH * NW
    NH = _round_up(N + 16, NW * 16)
    COLS = NH // NW
    IT = CH // 16
    NG = CH // _G

    def body_k1(x_hbm, idxi_hbm, idxj_hbm, local_hbm, hist_hbm, data_hbm,
                idxi_v, idxj_v, counts_v, local_v, rows0, rows1,
                semg0, semg1, semw0, semw1):
        w = _wid(NC)
        pltpu.sync_copy(idxi_hbm.at[pl.ds(w * CH, CH)], idxi_v)
        prologue, ring = _gather_ring(
            x_hbm, data_hbm, idxi_v, (rows0, rows1), (semg0, semg1),
            (semw0, semw1), w * CH, 0, NG)
        prologue()
        pltpu.sync_copy(idxj_hbm.at[pl.ds(w * CH, CH)], idxj_v)

        def zero(j, c):
            counts_v[pl.ds(j * 16, 16)] = jnp.zeros((16,), jnp.int32)
            return c
        lax.fori_loop(jnp.int32(0), jnp.int32(NH // 16), zero, 0)

        def step(i, c):
            v = idxj_v[pl.ds(i * 16, 16)]
            cnt, last = plsc.scan_count(v)
            base = plsc.load_gather(counts_v, [v])
            local_v[pl.ds(i * 16, 16)] = base + cnt - 1
            plsc.store_scatter(counts_v, [v], base + cnt, mask=last)
            return c
        lax.fori_loop(jnp.int32(0), jnp.int32(IT), step, 0)

        ring()
        pltpu.sync_copy(local_v, local_hbm.at[pl.ds(w * CH, CH)])
        pltpu.sync_copy(counts_v, hist_hbm.at[w])

    kernel_1 = pl.kernel(
        body_k1,
        out_type=(jax.ShapeDtypeStruct((EP,), jnp.int32),
                  jax.ShapeDtypeStruct((NW, NH), jnp.int32),
                  jax.ShapeDtypeStruct((EP, D), jnp.float32)),
        mesh=mesh,
        compiler_params=_CP,
        scratch_types=[pltpu.VMEM((CH,), jnp.int32),
                       pltpu.VMEM((CH,), jnp.int32),
                       pltpu.VMEM((NH,), jnp.int32),
                       pltpu.VMEM((CH,), jnp.int32),
                       pltpu.VMEM((_G, D), jnp.float32),
                       pltpu.VMEM((_G, D), jnp.float32),
                       pltpu.SemaphoreType.DMA, pltpu.SemaphoreType.DMA,
                       pltpu.SemaphoreType.DMA, pltpu.SemaphoreType.DMA],
        name="mp_count_gather1",
    )

    def body_k2(hist_hbm, offs_hbm, blk_v):
        w = _wid(NC)
        pltpu.sync_copy(hist_hbm.at[:, pl.ds(w * COLS, COLS)], blk_v)
        for j in range(COLS // 16):
            def step(t, acc):
                new = blk_v[t, pl.ds(j * 16, 16)]
                blk_v[t, pl.ds(j * 16, 16)] = acc
                return acc + new
            lax.fori_loop(jnp.int32(0), jnp.int32(NW), step,
                          jnp.zeros((16,), jnp.int32))
        pltpu.sync_copy(blk_v, offs_hbm.at[:, pl.ds(w * COLS, COLS)])

    kernel_2 = pl.kernel(
        body_k2,
        out_type=jax.ShapeDtypeStruct((NW, NH), jnp.int32),
        mesh=mesh,
        compiler_params=_CP,
        scratch_types=[pltpu.VMEM((NW, COLS), jnp.int32)],
        name="mp_prefix",
    )

    def body_k3(idxj_hbm, local_hbm, offs_hbm, out_hbm,
                idxj_v, local_v, offrow_v, out_v):
        w = _wid(NC)
        pltpu.sync_copy(offs_hbm.at[w], offrow_v)
        pltpu.sync_copy(idxj_hbm.at[pl.ds(w * CH, CH)], idxj_v)
        pltpu.sync_copy(local_hbm.at[pl.ds(w * CH, CH)], local_v)

        def step(i, c):
            v = idxj_v[pl.ds(i * 16, 16)]
            out_v[pl.ds(i * 16, 16)] = (
                local_v[pl.ds(i * 16, 16)] + plsc.load_gather(offrow_v, [v]))
            return c
        lax.fori_loop(jnp.int32(0), jnp.int32(IT), step, 0)
        pltpu.sync_copy(out_v, out_hbm.at[pl.ds(w * CH, CH)])

    kernel_3 = pl.kernel(
        body_k3,
        out_type=jax.ShapeDtypeStruct((EP,), jnp.int32),
        mesh=mesh,
        compiler_params=_CP,
        scratch_types=[pltpu.VMEM((CH,), jnp.int32),
                       pltpu.VMEM((CH,), jnp.int32),
                       pltpu.VMEM((NH,), jnp.int32),
                       pltpu.VMEM((CH,), jnp.int32)],
        name="mp_combine",
    )
    return kernel_1, kernel_2, kernel_3, CH, EP, NH


def kernel(x, edge_index):
    N, D = x.shape
    E = edge_index.shape[0]
    info = plsc.get_sparse_core_info()
    NC, NS = info.num_cores, info.num_subcores
    NW = NC * NS
    mesh = plsc.VectorSubcoreMesh(core_axis_name="c", subcore_axis_name="s")

    kernel_1, kernel_2, kernel_3, CH, EP, NH = _make_kernels(
        E, N, D, NW, NC, mesh)

    idxi = edge_index[:, 0].astype(jnp.int32)
    idxj = edge_index[:, 1].astype(jnp.int32)
    pad = EP - E
    idxi_p = jnp.concatenate([idxi, jnp.zeros((pad,), jnp.int32)])
    idxj_p = jnp.concatenate([idxj, jnp.full((pad,), N, jnp.int32)])

    local, hist, data = kernel_1(x, idxi_p, idxj_p)
    offs = kernel_2(hist)
    idx_el = kernel_3(idxj_p, local, offs)

    data = data[:E]
    idxs = jnp.concatenate(
        [edge_index[:, 1][None, :], idx_el[:E].astype(edge_index.dtype)[None, :]],
        axis=0)
    return data, idxs


# speedup vs baseline: 50.5419x; 1.9283x over previous
import jax
import jax.numpy as jnp
from jax import lax
from jax.experimental import pallas as pl
from jax.experimental.pallas import tpu as pltpu
from jax.experimental.pallas import tpu_sc as plsc


def _round_up(a, b):
    return (a + b - 1) // b * b


_CP = pltpu.CompilerParams(needs_layout_passes=False, use_tc_tiling_on_sc=False)

_G = 128


def _wid(nc):
    return lax.axis_index("s") * nc + lax.axis_index("c")


def _gather_ring(x_hbm, data_hbm, idxi_v, rows, semg, semw, base, g_hi, ng):
    def fire_gather(g, b):
        pltpu.async_copy(
            x_hbm.at[idxi_v.at[pl.ds(g * _G, _G)]], rows[b], semg[b])

    def wait_gather(b):
        pltpu.make_async_copy(
            x_hbm.at[idxi_v.at[pl.ds(0, _G)]], rows[b], semg[b]).wait()

    def wb_slice(g):
        return data_hbm.at[pl.ds(base + g * _G, _G)]

    def prologue():
        @pl.when(jnp.int32(0) < g_hi)
        def _():
            fire_gather(jnp.int32(0), 0)
        @pl.when(jnp.int32(1) < g_hi)
        def _():
            fire_gather(jnp.int32(1), 1)

    def ring():
        def step(g2, c):
            for b in range(2):
                g = g2 * 2 + b
                @pl.when(g < g_hi)
                def _():
                    wait_gather(b)
                    pltpu.async_copy(rows[b], wb_slice(g), semw[b])
                    pltpu.make_async_copy(rows[b], wb_slice(g), semw[b]).wait()
                    @pl.when(g + 2 < g_hi)
                    def _():
                        fire_gather(g + 2, b)
            return c
        lax.fori_loop(jnp.int32(0), jnp.int32((ng + 1) // 2), step, 0)

    return prologue, ring


def _make_kernels(E, N, D, NW, NC, mesh):
    CH = _round_up(-(-E // NW), 2 * _G)
    EP = CH * NW
    NH = _round_up(N + 16, NW * 16)
    COLS = NH // NW
    IT = CH // 16
    NG = CH // _G

    def body_k1(x_hbm, idxi_hbm, idxj_hbm, local_hbm, hist_hbm, data_hbm,
                idxi_v, idxj_v, counts_v, local_v, rows0, rows1,
                semg0, semg1, semw0, semw1):
        w = _wid(NC)
        pltpu.sync_copy(idxi_hbm.at[pl.ds(w * CH, CH)], idxi_v)
        ng_w = jnp.clip(jnp.int32(E // _G) - w * NG, 0, NG)
        prologue, ring = _gather_ring(
            x_hbm, data_hbm, idxi_v, (rows0, rows1), (semg0, semg1),
            (semw0, semw1), w * CH, ng_w, NG)
        prologue()
        pltpu.sync_copy(idxj_hbm.at[pl.ds(w * CH, CH)], idxj_v)

        def zero(j, c):
            counts_v[pl.ds(j * 16, 16)] = jnp.zeros((16,), jnp.int32)
            return c
        lax.fori_loop(jnp.int32(0), jnp.int32(NH // 16), zero, 0)

        def step(i, c):
            v = idxj_v[pl.ds(i * 16, 16)]
            cnt, last = plsc.scan_count(v)
            base = plsc.load_gather(counts_v, [v])
            local_v[pl.ds(i * 16, 16)] = base + cnt - 1
            plsc.store_scatter(counts_v, [v], base + cnt, mask=last)
            return c
        lax.fori_loop(jnp.int32(0), jnp.int32(IT), step, 0)

        ring()
        pltpu.sync_copy(local_v, local_hbm.at[pl.ds(w * CH, CH)])
        pltpu.sync_copy(counts_v, hist_hbm.at[w])

    kernel_1 = pl.kernel(
        body_k1,
        out_type=(jax.ShapeDtypeStruct((EP,), jnp.int32),
                  jax.ShapeDtypeStruct((NW, NH), jnp.int32),
                  jax.ShapeDtypeStruct((E, D), jnp.float32)),
        mesh=mesh,
        compiler_params=_CP,
        scratch_types=[pltpu.VMEM((CH,), jnp.int32),
                       pltpu.VMEM((CH,), jnp.int32),
                       pltpu.VMEM((NH,), jnp.int32),
                       pltpu.VMEM((CH,), jnp.int32),
                       pltpu.VMEM((_G, D), jnp.float32),
                       pltpu.VMEM((_G, D), jnp.float32),
                       pltpu.SemaphoreType.DMA, pltpu.SemaphoreType.DMA,
                       pltpu.SemaphoreType.DMA, pltpu.SemaphoreType.DMA],
        name="mp_count_gather1",
    )

    def body_k2(hist_hbm, offs_hbm, blk_v):
        w = _wid(NC)
        pltpu.sync_copy(hist_hbm.at[:, pl.ds(w * COLS, COLS)], blk_v)
        for j in range(COLS // 16):
            def step(t, acc):
                new = blk_v[t, pl.ds(j * 16, 16)]
                blk_v[t, pl.ds(j * 16, 16)] = acc
                return acc + new
            lax.fori_loop(jnp.int32(0), jnp.int32(NW), step,
                          jnp.zeros((16,), jnp.int32))
        pltpu.sync_copy(blk_v, offs_hbm.at[:, pl.ds(w * COLS, COLS)])

    kernel_2 = pl.kernel(
        body_k2,
        out_type=jax.ShapeDtypeStruct((NW, NH), jnp.int32),
        mesh=mesh,
        compiler_params=_CP,
        scratch_types=[pltpu.VMEM((NW, COLS), jnp.int32)],
        name="mp_prefix",
    )

    def body_k3(idxj_hbm, local_hbm, offs_hbm, out_hbm,
                idxj_v, local_v, offrow_v, out_v):
        w = _wid(NC)
        pltpu.sync_copy(offs_hbm.at[w], offrow_v)
        pltpu.sync_copy(idxj_hbm.at[pl.ds(w * CH, CH)], idxj_v)
        pltpu.sync_copy(local_hbm.at[pl.ds(w * CH, CH)], local_v)

        def step(i, c):
            v = idxj_v[pl.ds(i * 16, 16)]
            out_v[pl.ds(i * 16, 16)] = (
                local_v[pl.ds(i * 16, 16)] + plsc.load_gather(offrow_v, [v]))
            return c
        lax.fori_loop(jnp.int32(0), jnp.int32(IT), step, 0)
        pltpu.sync_copy(out_v, out_hbm.at[pl.ds(w * CH, CH)])

    kernel_3 = pl.kernel(
        body_k3,
        out_type=jax.ShapeDtypeStruct((EP,), jnp.int32),
        mesh=mesh,
        compiler_params=_CP,
        scratch_types=[pltpu.VMEM((CH,), jnp.int32),
                       pltpu.VMEM((CH,), jnp.int32),
                       pltpu.VMEM((NH,), jnp.int32),
                       pltpu.VMEM((CH,), jnp.int32)],
        name="mp_combine",
    )
    return kernel_1, kernel_2, kernel_3, CH, EP, NH


def kernel(x, edge_index):
    N, D = x.shape
    E = edge_index.shape[0]
    info = plsc.get_sparse_core_info()
    NC, NS = info.num_cores, info.num_subcores
    NW = NC * NS
    mesh = plsc.VectorSubcoreMesh(core_axis_name="c", subcore_axis_name="s")

    kernel_1, kernel_2, kernel_3, CH, EP, NH = _make_kernels(
        E, N, D, NW, NC, mesh)

    idxi = edge_index[:, 0].astype(jnp.int32)
    idxj = edge_index[:, 1].astype(jnp.int32)
    pad = EP - E
    idxi_p = jnp.concatenate([idxi, jnp.zeros((pad,), jnp.int32)])
    idxj_p = jnp.concatenate([idxj, jnp.full((pad,), N, jnp.int32)])

    local, hist, data = kernel_1(x, idxi_p, idxj_p)
    offs = kernel_2(hist)
    idx_el = kernel_3(idxj_p, local, offs)
    idxs = jnp.concatenate(
        [edge_index[:, 1][None, :], idx_el[:E].astype(edge_index.dtype)[None, :]],
        axis=0)
    return data, idxs


# speedup vs baseline: 99.0465x; 1.9597x over previous
import jax
import jax.numpy as jnp
from jax import lax
from jax.experimental import pallas as pl
from jax.experimental.pallas import tpu as pltpu
from jax.experimental.pallas import tpu_sc as plsc


def _round_up(a, b):
    return (a + b - 1) // b * b


_CP = pltpu.CompilerParams(needs_layout_passes=False, use_tc_tiling_on_sc=False)
_CPT = pltpu.CompilerParams(needs_layout_passes=False, use_tc_tiling_on_sc=True)

_G = 128


def _wid(nc):
    return lax.axis_index("s") * nc + lax.axis_index("c")


def _gather_ring(x_hbm, data_hbm, idxi_v, rows, semg, semw, base, g_hi, ng):
    def fire_gather(g, b):
        pltpu.async_copy(
            x_hbm.at[idxi_v.at[pl.ds(g * _G, _G)]], rows[b], semg[b])

    def wait_gather(b):
        pltpu.make_async_copy(
            x_hbm.at[idxi_v.at[pl.ds(0, _G)]], rows[b], semg[b]).wait()

    def wb_slice(g):
        return data_hbm.at[pl.ds(base + g * _G, _G)]

    def prologue():
        @pl.when(jnp.int32(0) < g_hi)
        def _():
            fire_gather(jnp.int32(0), 0)
        @pl.when(jnp.int32(1) < g_hi)
        def _():
            fire_gather(jnp.int32(1), 1)

    def ring():
        def step(g2, c):
            for b in range(2):
                g = g2 * 2 + b
                @pl.when(g < g_hi)
                def _():
                    wait_gather(b)
                    pltpu.async_copy(rows[b], wb_slice(g), semw[b])
                    pltpu.make_async_copy(rows[b], wb_slice(g), semw[b]).wait()
                    @pl.when(g + 2 < g_hi)
                    def _():
                        fire_gather(g + 2, b)
            return c
        lax.fori_loop(jnp.int32(0), jnp.int32((ng + 1) // 2), step, 0)

    return prologue, ring


def _make_kernels(E, N, D, NW, NC, mesh):
    CH = _round_up(-(-E // NW), 2 * _G)
    EP = CH * NW
    NH = _round_up(N + 16, NW * 16)
    COLS = NH // NW
    IT = CH // 16
    NG = CH // _G

    def body_k1(x_hbm, idxi_hbm, idxj_hbm, local_hbm, hist_hbm, data_hbm,
                idxi_v, idxj_v, counts_v, local_v, rows0, rows1,
                semg0, semg1, semw0, semw1):
        w = _wid(NC)
        pltpu.sync_copy(idxi_hbm.at[pl.ds(w * CH, CH)], idxi_v)
        ng_w = jnp.clip(jnp.int32(E // _G) - w * NG, 0, NG)
        prologue, ring = _gather_ring(
            x_hbm, data_hbm, idxi_v, (rows0, rows1), (semg0, semg1),
            (semw0, semw1), w * CH, ng_w, NG)
        prologue()
        pltpu.sync_copy(idxj_hbm.at[pl.ds(w * CH, CH)], idxj_v)

        def zero(j, c):
            counts_v[pl.ds(j * 16, 16)] = jnp.zeros((16,), jnp.int32)
            return c
        lax.fori_loop(jnp.int32(0), jnp.int32(NH // 16), zero, 0)

        def step(i, c):
            v = idxj_v[pl.ds(i * 16, 16)]
            cnt, last = plsc.scan_count(v)
            base = plsc.load_gather(counts_v, [v])
            local_v[pl.ds(i * 16, 16)] = base + cnt - 1
            plsc.store_scatter(counts_v, [v], base + cnt, mask=last)
            return c
        lax.fori_loop(jnp.int32(0), jnp.int32(IT), step, 0)

        ring()
        pltpu.sync_copy(local_v, local_hbm.at[pl.ds(w * CH, CH)])
        pltpu.sync_copy(counts_v, hist_hbm.at[w])

    kernel_1 = pl.kernel(
        body_k1,
        out_type=(jax.ShapeDtypeStruct((EP,), jnp.int32),
                  jax.ShapeDtypeStruct((NW, NH), jnp.int32),
                  jax.ShapeDtypeStruct((E, D), jnp.float32)),
        mesh=mesh,
        compiler_params=_CPT,
        scratch_types=[pltpu.VMEM((CH,), jnp.int32),
                       pltpu.VMEM((CH,), jnp.int32),
                       pltpu.VMEM((NH,), jnp.int32),
                       pltpu.VMEM((CH,), jnp.int32),
                       pltpu.VMEM((_G, D), jnp.float32),
                       pltpu.VMEM((_G, D), jnp.float32),
                       pltpu.SemaphoreType.DMA, pltpu.SemaphoreType.DMA,
                       pltpu.SemaphoreType.DMA, pltpu.SemaphoreType.DMA],
        name="mp_count_gather1",
    )

    def body_k2(hist_hbm, offs_hbm, blk_v):
        w = _wid(NC)
        pltpu.sync_copy(hist_hbm.at[:, pl.ds(w * COLS, COLS)], blk_v)
        for j in range(COLS // 16):
            def step(t, acc):
                new = blk_v[t, pl.ds(j * 16, 16)]
                blk_v[t, pl.ds(j * 16, 16)] = acc
                return acc + new
            lax.fori_loop(jnp.int32(0), jnp.int32(NW), step,
                          jnp.zeros((16,), jnp.int32))
        pltpu.sync_copy(blk_v, offs_hbm.at[:, pl.ds(w * COLS, COLS)])

    kernel_2 = pl.kernel(
        body_k2,
        out_type=jax.ShapeDtypeStruct((NW, NH), jnp.int32),
        mesh=mesh,
        compiler_params=_CP,
        scratch_types=[pltpu.VMEM((NW, COLS), jnp.int32)],
        name="mp_prefix",
    )

    def body_k3(idxj_hbm, local_hbm, offs_hbm, out_hbm,
                idxj_v, local_v, offrow_v, out_v):
        w = _wid(NC)
        pltpu.sync_copy(offs_hbm.at[w], offrow_v)
        pltpu.sync_copy(idxj_hbm.at[pl.ds(w * CH, CH)], idxj_v)
        pltpu.sync_copy(local_hbm.at[pl.ds(w * CH, CH)], local_v)

        def step(i, c):
            v = idxj_v[pl.ds(i * 16, 16)]
            out_v[pl.ds(i * 16, 16)] = (
                local_v[pl.ds(i * 16, 16)] + plsc.load_gather(offrow_v, [v]))
            return c
        lax.fori_loop(jnp.int32(0), jnp.int32(IT), step, 0)
        pltpu.sync_copy(out_v, out_hbm.at[pl.ds(w * CH, CH)])

    kernel_3 = pl.kernel(
        body_k3,
        out_type=jax.ShapeDtypeStruct((EP,), jnp.int32),
        mesh=mesh,
        compiler_params=_CP,
        scratch_types=[pltpu.VMEM((CH,), jnp.int32),
                       pltpu.VMEM((CH,), jnp.int32),
                       pltpu.VMEM((NH,), jnp.int32),
                       pltpu.VMEM((CH,), jnp.int32)],
        name="mp_combine",
    )
    return kernel_1, kernel_2, kernel_3, CH, EP, NH


def kernel(x, edge_index):
    N, D = x.shape
    E = edge_index.shape[0]
    info = plsc.get_sparse_core_info()
    NC, NS = info.num_cores, info.num_subcores
    NW = NC * NS
    mesh = plsc.VectorSubcoreMesh(core_axis_name="c", subcore_axis_name="s")

    kernel_1, kernel_2, kernel_3, CH, EP, NH = _make_kernels(
        E, N, D, NW, NC, mesh)

    idxi = edge_index[:, 0].astype(jnp.int32)
    idxj = edge_index[:, 1].astype(jnp.int32)
    pad = EP - E
    idxi_p = jnp.concatenate([idxi, jnp.zeros((pad,), jnp.int32)])
    idxj_p = jnp.concatenate([idxj, jnp.full((pad,), N, jnp.int32)])

    local, hist, data = kernel_1(x, idxi_p, idxj_p)
    offs = kernel_2(hist)
    idx_el = kernel_3(idxj_p, local, offs)
    idxs = jnp.concatenate(
        [edge_index[:, 1][None, :], idx_el[:E].astype(edge_index.dtype)[None, :]],
        axis=0)
    return data, idxs


# speedup vs baseline: 101.8806x; 1.0286x over previous
import jax
import jax.numpy as jnp
from jax import lax
from jax.experimental import pallas as pl
from jax.experimental.pallas import tpu as pltpu
from jax.experimental.pallas import tpu_sc as plsc


def _round_up(a, b):
    return (a + b - 1) // b * b


_CP = pltpu.CompilerParams(needs_layout_passes=False, use_tc_tiling_on_sc=False)
_CPT = pltpu.CompilerParams(needs_layout_passes=False, use_tc_tiling_on_sc=True)

_G = 128


def _wid(nc):
    return lax.axis_index("s") * nc + lax.axis_index("c")


def _gather_ring(x_hbm, data_hbm, idxi_v, rows, semg, semw, base, g_hi, ng):
    def fire_gather(g, b):
        pltpu.async_copy(
            x_hbm.at[idxi_v.at[pl.ds(g * _G, _G)]], rows[b], semg[b])

    def wait_gather(b):
        pltpu.make_async_copy(
            x_hbm.at[idxi_v.at[pl.ds(0, _G)]], rows[b], semg[b]).wait()

    def wb_slice(g):
        return data_hbm.at[pl.ds(base + g * _G, _G)]

    def prologue():
        @pl.when(jnp.int32(0) < g_hi)
        def _():
            fire_gather(jnp.int32(0), 0)
        @pl.when(jnp.int32(1) < g_hi)
        def _():
            fire_gather(jnp.int32(1), 1)

    def ring(interleave=None):
        def step(g2, c):
            if interleave is not None:
                interleave(g2)
            for b in range(2):
                g = g2 * 2 + b
                @pl.when(g < g_hi)
                def _():
                    wait_gather(b)
                    pltpu.async_copy(rows[b], wb_slice(g), semw[b])
                    pltpu.make_async_copy(rows[b], wb_slice(g), semw[b]).wait()
                    @pl.when(g + 2 < g_hi)
                    def _():
                        fire_gather(g + 2, b)
            return c
        lax.fori_loop(jnp.int32(0), jnp.int32((ng + 1) // 2), step, 0)

    return prologue, ring


def _make_kernels(E, N, D, NW, NC, mesh):
    CH = _round_up(-(-E // NW), 2 * _G)
    EP = CH * NW
    NH = _round_up(N + 16, NW * 16)
    COLS = NH // NW
    IT = CH // 16
    NG = CH // _G

    def body_k1(x_hbm, idxi_hbm, idxj_hbm, local_hbm, hist_hbm, data_hbm,
                idxi_v, idxj_v, counts_v, local_v, rows0, rows1,
                semg0, semg1, semw0, semw1):
        w = _wid(NC)
        pltpu.sync_copy(idxi_hbm.at[pl.ds(w * CH, CH)], idxi_v)
        ng_w = jnp.clip(jnp.int32(E // _G) - w * NG, 0, NG)
        prologue, ring = _gather_ring(
            x_hbm, data_hbm, idxi_v, (rows0, rows1), (semg0, semg1),
            (semw0, semw1), w * CH, ng_w, NG)
        prologue()
        pltpu.sync_copy(idxj_hbm.at[pl.ds(w * CH, CH)], idxj_v)

        def zero(j, c):
            counts_v[pl.ds(j * 16, 16)] = jnp.zeros((16,), jnp.int32)
            return c
        lax.fori_loop(jnp.int32(0), jnp.int32(NH // 16), zero, 0)

        def count_step(i, c):
            v = idxj_v[pl.ds(i * 16, 16)]
            cnt, last = plsc.scan_count(v)
            base = plsc.load_gather(counts_v, [v])
            local_v[pl.ds(i * 16, 16)] = base + cnt - 1
            plsc.store_scatter(counts_v, [v], base + cnt, mask=last)
            return c

        IPS = IT // ((NG + 1) // 2)

        def interleave(g2):
            lax.fori_loop(g2 * IPS, (g2 + 1) * IPS, count_step, 0)

        ring(interleave)
        lax.fori_loop(jnp.int32(IPS * ((NG + 1) // 2)), jnp.int32(IT),
                      count_step, 0)
        pltpu.sync_copy(local_v, local_hbm.at[pl.ds(w * CH, CH)])
        pltpu.sync_copy(counts_v, hist_hbm.at[w])

    kernel_1 = pl.kernel(
        body_k1,
        out_type=(jax.ShapeDtypeStruct((EP,), jnp.int32),
                  jax.ShapeDtypeStruct((NW, NH), jnp.int32),
                  jax.ShapeDtypeStruct((E, D), jnp.float32)),
        mesh=mesh,
        compiler_params=_CPT,
        scratch_types=[pltpu.VMEM((CH,), jnp.int32),
                       pltpu.VMEM((CH,), jnp.int32),
                       pltpu.VMEM((NH,), jnp.int32),
                       pltpu.VMEM((CH,), jnp.int32),
                       pltpu.VMEM((_G, D), jnp.float32),
                       pltpu.VMEM((_G, D), jnp.float32),
                       pltpu.SemaphoreType.DMA, pltpu.SemaphoreType.DMA,
                       pltpu.SemaphoreType.DMA, pltpu.SemaphoreType.DMA],
        name="mp_count_gather1",
    )

    def body_k2(hist_hbm, offs_hbm, blk_v):
        w = _wid(NC)
        pltpu.sync_copy(hist_hbm.at[:, pl.ds(w * COLS, COLS)], blk_v)
        for j in range(COLS // 16):
            def step(t, acc):
                new = blk_v[t, pl.ds(j * 16, 16)]
                blk_v[t, pl.ds(j * 16, 16)] = acc
                return acc + new
            lax.fori_loop(jnp.int32(0), jnp.int32(NW), step,
                          jnp.zeros((16,), jnp.int32))
        pltpu.sync_copy(blk_v, offs_hbm.at[:, pl.ds(w * COLS, COLS)])

    kernel_2 = pl.kernel(
        body_k2,
        out_type=jax.ShapeDtypeStruct((NW, NH), jnp.int32),
        mesh=mesh,
        compiler_params=_CP,
        scratch_types=[pltpu.VMEM((NW, COLS), jnp.int32)],
        name="mp_prefix",
    )

    def body_k3(idxj_hbm, local_hbm, offs_hbm, out_hbm,
                idxj_v, local_v, offrow_v, out_v):
        w = _wid(NC)
        pltpu.sync_copy(offs_hbm.at[w], offrow_v)
        pltpu.sync_copy(idxj_hbm.at[pl.ds(w * CH, CH)], idxj_v)
        pltpu.sync_copy(local_hbm.at[pl.ds(w * CH, CH)], local_v)

        def step(i, c):
            v = idxj_v[pl.ds(i * 16, 16)]
            out_v[pl.ds(i * 16, 16)] = (
                local_v[pl.ds(i * 16, 16)] + plsc.load_gather(offrow_v, [v]))
            return c
        lax.fori_loop(jnp.int32(0), jnp.int32(IT), step, 0)
        pltpu.sync_copy(out_v, out_hbm.at[pl.ds(w * CH, CH)])

    kernel_3 = pl.kernel(
        body_k3,
        out_type=jax.ShapeDtypeStruct((EP,), jnp.int32),
        mesh=mesh,
        compiler_params=_CP,
        scratch_types=[pltpu.VMEM((CH,), jnp.int32),
                       pltpu.VMEM((CH,), jnp.int32),
                       pltpu.VMEM((NH,), jnp.int32),
                       pltpu.VMEM((CH,), jnp.int32)],
        name="mp_combine",
    )
    return kernel_1, kernel_2, kernel_3, CH, EP, NH


def kernel(x, edge_index):
    N, D = x.shape
    E = edge_index.shape[0]
    info = plsc.get_sparse_core_info()
    NC, NS = info.num_cores, info.num_subcores
    NW = NC * NS
    mesh = plsc.VectorSubcoreMesh(core_axis_name="c", subcore_axis_name="s")

    kernel_1, kernel_2, kernel_3, CH, EP, NH = _make_kernels(
        E, N, D, NW, NC, mesh)

    idxi = edge_index[:, 0].astype(jnp.int32)
    idxj = edge_index[:, 1].astype(jnp.int32)
    pad = EP - E
    idxi_p = jnp.concatenate([idxi, jnp.zeros((pad,), jnp.int32)])
    idxj_p = jnp.concatenate([idxj, jnp.full((pad,), N, jnp.int32)])

    local, hist, data = kernel_1(x, idxi_p, idxj_p)
    offs = kernel_2(hist)
    idx_el = kernel_3(idxj_p, local, offs)
    idxs = jnp.concatenate(
        [edge_index[:, 1][None, :], idx_el[:E].astype(edge_index.dtype)[None, :]],
        axis=0)
    return data, idxs
